# baseline (speedup 1.0000x reference)
import functools

import jax
import jax.numpy as jnp
from jax import lax
from jax.experimental import pallas as pl
from jax.experimental.pallas import tpu as pltpu
from jax.experimental.pallas import tpu_sc as plsc

B = 16384
K = 20
D = 32
NW = 32
ROWS_W = B // NW
C = 64
NCHUNK = ROWS_W // C
NIDX_ROWS = C * K // 128


def _body(uidx_hbm, pidx_hbm, nidx_hbm, utab, itab, pos_out, neg_out,
          uidx_v, pidx_v, nidx_v, urows_v, prows_v, nrows_v, pout_v, nout_v,
          sem_a, sem_b, sem_i):
    cid = lax.axis_index("c")
    sid = lax.axis_index("s")
    wid = sid * 2 + cid
    l16 = lax.iota(jnp.int32, 16)
    sems = [sem_a, sem_b]

    wbase = wid * ROWS_W
    ih = [pltpu.async_copy(uidx_hbm.at[pl.ds(wbase, ROWS_W)], uidx_v, sem_i),
          pltpu.async_copy(pidx_hbm.at[pl.ds(wbase, ROWS_W)], pidx_v, sem_i),
          pltpu.async_copy(nidx_hbm.at[pl.ds(wbase * K, ROWS_W * K)],
                           nidx_v, sem_i)]
    for h in ih:
        h.wait()

    def stage_and_fire(c):
        p = c % 2
        hs = [pltpu.async_copy(utab.at[uidx_v.at[pl.ds(c * C, C)]],
                               urows_v.at[p], sems[p]),
              pltpu.async_copy(itab.at[pidx_v.at[pl.ds(c * C, C)]],
                               prows_v.at[p], sems[p])]
        for j in range(NIDX_ROWS):
            hs.append(pltpu.async_copy(
                itab.at[nidx_v.at[pl.ds(c * C * K + j * 128, 128)]],
                nrows_v.at[p, pl.ds(j * 128, 128)], sems[p]))
        return hs

    lane15 = l16 == 15
    zero16 = jnp.full((16,), 0, jnp.int32)

    def compute(c):
        p = c % 2
        base = wid * ROWS_W + c * C
        urows = urows_v.at[p]
        prows = prows_v.at[p]
        nrows = nrows_v.at[p]

        NI = 8

        def row_body(rq, rcarry):
            rs = [rq * NI + i for i in range(NI)]
            us = [(urows[r, pl.ds(0, 16)], urows[r, pl.ds(16, 16)])
                  for r in rs]
            ps = [(prows[r, pl.ds(0, 16)], prows[r, pl.ds(16, 16)])
                  for r in rs]
            ridx = [zero16 + r for r in rs]
            cps = [plsc.cumsum(us[i][0] * ps[i][0] + us[i][1] * ps[i][1])
                   for i in range(NI)]
            for i in range(NI):
                plsc.store_scatter(pout_v, [ridx[i]], cps[i], mask=lane15)
            for k in range(K):
                ns = [(nrows[rs[i] * K + k, pl.ds(0, 16)],
                       nrows[rs[i] * K + k, pl.ds(16, 16)])
                      for i in range(NI)]
                cs = [plsc.cumsum(us[i][0] * ns[i][0] + us[i][1] * ns[i][1])
                      for i in range(NI)]
                for i in range(NI):
                    plsc.store_scatter(nout_v, [ridx[i], zero16 + k], cs[i],
                                       mask=lane15)
            return rcarry

        lax.fori_loop(0, C // NI, row_body, 0)
        pltpu.sync_copy(pout_v, pos_out.at[pl.ds(base, C)])
        pltpu.sync_copy(nout_v, neg_out.at[pl.ds(base, C)])

    hs = stage_and_fire(0)
    for c in range(NCHUNK):
        nxt = stage_and_fire(c + 1) if c + 1 < NCHUNK else []
        for h in hs:
            h.wait()
        compute(c)
        hs = nxt


@jax.jit
def _sc_call(user, pos_item, neg_flat, utab, itab):
    mesh = plsc.VectorSubcoreMesh(core_axis_name="c", subcore_axis_name="s")
    kfn = functools.partial(
        pl.kernel,
        out_type=[jax.ShapeDtypeStruct((B,), jnp.float32),
                  jax.ShapeDtypeStruct((B, K), jnp.float32)],
        mesh=mesh,
        scratch_types=[
            pltpu.VMEM((ROWS_W,), jnp.int32),
            pltpu.VMEM((ROWS_W,), jnp.int32),
            pltpu.VMEM((ROWS_W * K,), jnp.int32),
            pltpu.VMEM((2, C, D), jnp.float32),
            pltpu.VMEM((2, C, D), jnp.float32),
            pltpu.VMEM((2, C * K, D), jnp.float32),
            pltpu.VMEM((C,), jnp.float32),
            pltpu.VMEM((C, K), jnp.float32),
            pltpu.SemaphoreType.DMA,
            pltpu.SemaphoreType.DMA,
            pltpu.SemaphoreType.DMA,
        ],
        compiler_params=pltpu.CompilerParams(needs_layout_passes=False,
                                             use_tc_tiling_on_sc=False),
    )(_body)
    return kfn(user, pos_item, neg_flat, utab, itab)


def kernel(user, pos_item, neg_items, user_table, item_table):
    user = user.astype(jnp.int32)
    pos_item = pos_item.astype(jnp.int32)
    neg_flat = neg_items.astype(jnp.int32).reshape(B * K)
    pos_s, neg_s = _sc_call(user, pos_item, neg_flat, user_table, item_table)
    return (pos_s, neg_s)
